# baseline (speedup 1.0000x reference)
import functools

import jax
import jax.numpy as jnp
from jax import lax
from jax.experimental import pallas as pl
from jax.experimental.pallas import tpu as pltpu
from jax.experimental.pallas import tpu_sc as plsc

DIM = 256
K = 8192
B = 16
N = 1024
BN = B * N

TN = 256
GRID = BN // TN


def _dist_argmax_body(x_ref, et_ref, sel_ref, dist_ref, idx_ref, e2_ref):
    i = pl.program_id(0)

    @pl.when(i == 0)
    def _():
        et = et_ref[...]
        e2_ref[0, :] = jnp.sum(et * et, axis=0)

    xb = x_ref[...]
    xe2 = lax.dot_general(
        xb + xb, et_ref[...],
        dimension_numbers=(((1,), (0,)), ((), ())),
        preferred_element_type=jnp.float32,
    )
    x2 = jnp.sum(xb * xb, axis=1, keepdims=True)
    dist = (xe2 - x2) - e2_ref[0, :][None, :]
    dist_ref[...] = dist
    m = jnp.max(dist, axis=1, keepdims=True)
    mask = (dist == m).astype(jnp.float32)
    red = lax.dot_general(
        mask, sel_ref[...],
        dimension_numbers=(((1,), (0,)), ((), ())),
        preferred_element_type=jnp.float32,
    )
    cnt = red[:, 0]
    idx_ref[0, 0, :] = (jnp.float32(K) - red[:, 1]).astype(jnp.int32)

    @pl.when(jnp.any(cnt > 1.5))
    def _():
        d = dist_ref[...]
        mm = jnp.max(d, axis=1, keepdims=True)
        ii = lax.broadcasted_iota(jnp.int32, d.shape, 1)
        idx_ref[0, 0, :] = jnp.min(jnp.where(d == mm, ii, jnp.int32(K)), axis=1)


_SEL_W = 128


def _dist_argmax(x_flat, embed_t):
    sel = jnp.zeros((K, _SEL_W), jnp.float32)
    sel = sel.at[:, 0].set(1.0)
    sel = sel.at[:, 1].set(jnp.float32(K) - jnp.arange(K, dtype=jnp.float32))
    return pl.pallas_call(
        _dist_argmax_body,
        grid=(GRID,),
        in_specs=[
            pl.BlockSpec((TN, DIM), lambda i: (i, 0)),
            pl.BlockSpec((DIM, K), lambda i: (0, 0)),
            pl.BlockSpec((K, _SEL_W), lambda i: (0, 0)),
        ],
        out_specs=[
            pl.BlockSpec((TN, K), lambda i: (i, 0)),
            pl.BlockSpec((1, 1, TN), lambda i: (i, 0, 0)),
        ],
        out_shape=[
            jax.ShapeDtypeStruct((BN, K), jnp.float32),
            jax.ShapeDtypeStruct((GRID, 1, TN), jnp.int32),
        ],
        scratch_shapes=[pltpu.VMEM((1, K), jnp.float32)],
    )(x_flat, embed_t, sel)


_CHUNK = 128


def _sc_gather(table, idx):
    info = plsc.get_sparse_core_info()
    nc, ns = info.num_cores, info.num_subcores
    b_per_w = BN // (nc * ns)
    nchunk = b_per_w // _CHUNK
    mesh = plsc.VectorSubcoreMesh(core_axis_name="c", subcore_axis_name="s")

    @functools.partial(
        pl.kernel,
        mesh=mesh,
        out_type=jax.ShapeDtypeStruct((BN, DIM), jnp.float32),
        scratch_types=[
            pltpu.VMEM((b_per_w,), jnp.int32),
            pltpu.VMEM((_CHUNK, DIM), jnp.float32),
            pltpu.SemaphoreType.DMA,
        ],
    )
    def gather_k(table_hbm, idx_hbm, out_hbm, idx_v, rows_v, sem):
        wid = lax.axis_index("s") * nc + lax.axis_index("c")
        base = wid * b_per_w
        pltpu.sync_copy(idx_hbm.at[pl.ds(base, b_per_w)], idx_v)
        for c in range(nchunk):
            pltpu.async_copy(
                table_hbm.at[idx_v.at[pl.ds(c * _CHUNK, _CHUNK)]], rows_v, sem
            ).wait()
            pltpu.sync_copy(rows_v, out_hbm.at[pl.ds(base + c * _CHUNK, _CHUNK)])

    return gather_k(table, idx)


def kernel(x, embed):
    x_flat = x.reshape(BN, DIM)
    embed2d = embed.reshape(K, DIM)
    dist, idx_blocks = _dist_argmax(x_flat, embed2d.T)
    idx_flat = idx_blocks.reshape(BN)
    quantize = _sc_gather(embed2d, idx_flat).reshape(B, N, DIM)
    return quantize, idx_flat.reshape(B, N), dist.reshape(1, BN, K)


# speedup vs baseline: 1.8526x; 1.8526x over previous
import functools

import jax
import jax.numpy as jnp
from jax import lax
from jax.experimental import pallas as pl
from jax.experimental.pallas import tpu as pltpu
from jax.experimental.pallas import tpu_sc as plsc

DIM = 256
K = 8192
B = 16
N = 1024
BN = B * N

TN = 256
GRID = BN // TN


def _dist_argmax_body(x_ref, et_ref, dist_ref, idx_ref, e2_ref, rev_ref):
    i = pl.program_id(0)

    @pl.when(i == 0)
    def _():
        et = et_ref[...]
        e2_ref[0, :] = jnp.sum(et * et, axis=0)
        ii = lax.broadcasted_iota(jnp.int32, (1, K), 1).astype(jnp.float32)
        rev_ref[...] = jnp.float32(K) - ii

    xb = x_ref[...]
    xe2 = lax.dot_general(
        xb + xb, et_ref[...],
        dimension_numbers=(((1,), (0,)), ((), ())),
        preferred_element_type=jnp.float32,
    )
    x2 = jnp.sum(xb * xb, axis=1, keepdims=True)
    dist = (xe2 - x2) - e2_ref[0, :][None, :]
    dist_ref[...] = dist
    m = jnp.max(dist, axis=1, keepdims=True)
    val = jnp.max(jnp.where(dist == m, rev_ref[0, :][None, :], jnp.float32(0)),
                  axis=1)
    idx_ref[0, 0, :] = (jnp.float32(K) - val).astype(jnp.int32)


def _dist_argmax(x_flat, embed_t):
    return pl.pallas_call(
        _dist_argmax_body,
        grid=(GRID,),
        in_specs=[
            pl.BlockSpec((TN, DIM), lambda i: (i, 0)),
            pl.BlockSpec((DIM, K), lambda i: (0, 0)),
        ],
        out_specs=[
            pl.BlockSpec((TN, K), lambda i: (i, 0)),
            pl.BlockSpec((1, 1, TN), lambda i: (i, 0, 0)),
        ],
        out_shape=[
            jax.ShapeDtypeStruct((BN, K), jnp.float32),
            jax.ShapeDtypeStruct((GRID, 1, TN), jnp.int32),
        ],
        scratch_shapes=[pltpu.VMEM((1, K), jnp.float32),
                        pltpu.VMEM((1, K), jnp.float32)],
    )(x_flat, embed_t)


_CHUNK = 128


def _sc_gather(table, idx):
    info = plsc.get_sparse_core_info()
    nc, ns = info.num_cores, info.num_subcores
    b_per_w = BN // (nc * ns)
    nchunk = b_per_w // _CHUNK
    mesh = plsc.VectorSubcoreMesh(core_axis_name="c", subcore_axis_name="s")

    @functools.partial(
        pl.kernel,
        mesh=mesh,
        out_type=jax.ShapeDtypeStruct((BN, DIM), jnp.float32),
        scratch_types=[
            pltpu.VMEM((b_per_w,), jnp.int32),
            pltpu.VMEM((_CHUNK, DIM), jnp.float32),
            pltpu.SemaphoreType.DMA,
        ],
    )
    def gather_k(table_hbm, idx_hbm, out_hbm, idx_v, rows_v, sem):
        wid = lax.axis_index("s") * nc + lax.axis_index("c")
        base = wid * b_per_w
        pltpu.sync_copy(idx_hbm.at[pl.ds(base, b_per_w)], idx_v)
        for c in range(nchunk):
            pltpu.async_copy(
                table_hbm.at[idx_v.at[pl.ds(c * _CHUNK, _CHUNK)]], rows_v, sem
            ).wait()
            pltpu.sync_copy(rows_v, out_hbm.at[pl.ds(base + c * _CHUNK, _CHUNK)])

    return gather_k(table, idx)


def kernel(x, embed):
    x_flat = x.reshape(BN, DIM)
    embed2d = embed.reshape(K, DIM)
    dist, idx_blocks = _dist_argmax(x_flat, embed2d.T)
    idx_flat = idx_blocks.reshape(BN)
    quantize = _sc_gather(embed2d, idx_flat).reshape(B, N, DIM)
    return quantize, idx_flat.reshape(B, N), dist.reshape(1, BN, K)


# speedup vs baseline: 1.9113x; 1.0317x over previous
import functools

import jax
import jax.numpy as jnp
from jax import lax
from jax.experimental import pallas as pl
from jax.experimental.pallas import tpu as pltpu
from jax.experimental.pallas import tpu_sc as plsc

DIM = 256
K = 8192
B = 16
N = 1024
BN = B * N

TN = 512
GRID = BN // TN
NCH = 16


def _dist_argmax_body(x_ref, et_ref, dist_ref, idx_ref, e2_ref, rev_ref):
    i = pl.program_id(0)

    @pl.when(i == 0)
    def _():
        et = et_ref[...]
        e2_ref[0, :] = jnp.sum(et * et, axis=0)
        ii = lax.broadcasted_iota(jnp.int32, (1, K), 1).astype(jnp.float32)
        rev_ref[...] = jnp.float32(K) - ii

    xb = x_ref[...]
    xb2 = xb + xb
    x2 = jnp.sum(xb * xb, axis=1, keepdims=True)
    cw = K // NCH
    ms = []
    for c in range(NCH):
        xe2_c = lax.dot_general(
            xb2, et_ref[:, c * cw:(c + 1) * cw],
            dimension_numbers=(((1,), (0,)), ((), ())),
            preferred_element_type=jnp.float32,
        )
        dist_c = (xe2_c - x2) - e2_ref[0, c * cw:(c + 1) * cw][None, :]
        dist_ref[:, c * cw:(c + 1) * cw] = dist_c
        ms.append(jnp.max(dist_c, axis=1, keepdims=True))
    m = functools.reduce(jnp.maximum, ms)
    vals = []
    for c in range(NCH):
        d_c = dist_ref[:, c * cw:(c + 1) * cw]
        rev_c = rev_ref[0, c * cw:(c + 1) * cw][None, :]
        vals.append(jnp.max(jnp.where(d_c == m, rev_c, jnp.float32(0)), axis=1))
    val = functools.reduce(jnp.maximum, vals)
    idx_ref[0, 0, :] = (jnp.float32(K) - val).astype(jnp.int32)


def _dist_argmax(x_flat, embed_t):
    return pl.pallas_call(
        _dist_argmax_body,
        grid=(GRID,),
        in_specs=[
            pl.BlockSpec((TN, DIM), lambda i: (i, 0)),
            pl.BlockSpec((DIM, K), lambda i: (0, 0)),
        ],
        out_specs=[
            pl.BlockSpec((TN, K), lambda i: (i, 0)),
            pl.BlockSpec((1, 1, TN), lambda i: (i, 0, 0)),
        ],
        out_shape=[
            jax.ShapeDtypeStruct((BN, K), jnp.float32),
            jax.ShapeDtypeStruct((GRID, 1, TN), jnp.int32),
        ],
        scratch_shapes=[pltpu.VMEM((1, K), jnp.float32),
                        pltpu.VMEM((1, K), jnp.float32)],
    )(x_flat, embed_t)


_CHUNK = 128


def _sc_gather(table, idx):
    info = plsc.get_sparse_core_info()
    nc, ns = info.num_cores, info.num_subcores
    b_per_w = BN // (nc * ns)
    nchunk = b_per_w // _CHUNK
    mesh = plsc.VectorSubcoreMesh(core_axis_name="c", subcore_axis_name="s")

    @functools.partial(
        pl.kernel,
        mesh=mesh,
        out_type=jax.ShapeDtypeStruct((BN, DIM), jnp.float32),
        scratch_types=[
            pltpu.VMEM((b_per_w,), jnp.int32),
            pltpu.VMEM((_CHUNK, DIM), jnp.float32),
            pltpu.SemaphoreType.DMA,
        ],
    )
    def gather_k(table_hbm, idx_hbm, out_hbm, idx_v, rows_v, sem):
        wid = lax.axis_index("s") * nc + lax.axis_index("c")
        base = wid * b_per_w
        pltpu.sync_copy(idx_hbm.at[pl.ds(base, b_per_w)], idx_v)
        for c in range(nchunk):
            pltpu.async_copy(
                table_hbm.at[idx_v.at[pl.ds(c * _CHUNK, _CHUNK)]], rows_v, sem
            ).wait()
            pltpu.sync_copy(rows_v, out_hbm.at[pl.ds(base + c * _CHUNK, _CHUNK)])

    return gather_k(table, idx)


def kernel(x, embed):
    x_flat = x.reshape(BN, DIM)
    embed2d = embed.reshape(K, DIM)
    dist, idx_blocks = _dist_argmax(x_flat, embed2d.T)
    idx_flat = idx_blocks.reshape(BN)
    quantize = _sc_gather(embed2d, idx_flat).reshape(B, N, DIM)
    return quantize, idx_flat.reshape(B, N), dist.reshape(1, BN, K)
